# Initial kernel scaffold; baseline (speedup 1.0000x reference)
import jax
import jax.numpy as jnp
from jax.experimental import pallas as pl


def kernel(x, edge_index, W1, b1, W2, b2):
    raise NotImplementedError("write your pallas kernel here")


# speedup vs baseline: 1.0072x; 1.0072x over previous
import jax
import jax.numpy as jnp
from jax.experimental import pallas as pl
from jax.experimental.pallas import tpu as pltpu

N_NODES = 10000
HIDDEN = 256
MOTIF = 64

_f32 = jnp.float32

R = 1000


def _tc_body(x_ref, ns_ref, deg_ref, w1_ref, b1_ref, w2_ref, b2_ref,
             o_ref, acc_ref):
    i = pl.program_id(0)
    nsum = ns_ref[...]
    deg = deg_ref[:, 0:1]
    nm = jnp.where(deg > 0, nsum / jnp.maximum(deg, 1.0),
                   jnp.zeros_like(nsum))
    xb = x_ref[...]
    h = (jnp.dot(xb, w1_ref[0:HIDDEN], preferred_element_type=_f32)
         + jnp.dot(nm, w1_ref[HIDDEN:2 * HIDDEN], preferred_element_type=_f32)
         + jnp.dot(xb * nm, w1_ref[2 * HIDDEN:3 * HIDDEN],
                   preferred_element_type=_f32)
         + b1_ref[...])
    h = jnp.maximum(h, 0.0)

    @pl.when(i == 0)
    def _():
        acc_ref[...] = jnp.zeros_like(acc_ref)

    acc_ref[...] += jnp.sum(h, axis=0, keepdims=True)

    @pl.when(i == pl.num_programs(0) - 1)
    def _():
        o_ref[...] = (jnp.dot(acc_ref[...] / N_NODES, w2_ref[...],
                              preferred_element_type=_f32) + b2_ref[...])


def _tc_mlp(x, ns, deg, W1, b1, W2, b2):
    grid = (N_NODES // R,)
    return pl.pallas_call(
        _tc_body,
        grid=grid,
        in_specs=[
            pl.BlockSpec((R, HIDDEN), lambda i: (i, 0)),
            pl.BlockSpec((R, HIDDEN), lambda i: (i, 0)),
            pl.BlockSpec((R, 1), lambda i: (i, 0)),
            pl.BlockSpec((3 * HIDDEN, HIDDEN), lambda i: (0, 0)),
            pl.BlockSpec((1, HIDDEN), lambda i: (0, 0)),
            pl.BlockSpec((HIDDEN, MOTIF), lambda i: (0, 0)),
            pl.BlockSpec((1, MOTIF), lambda i: (0, 0)),
        ],
        out_specs=pl.BlockSpec((1, MOTIF), lambda i: (0, 0)),
        out_shape=jax.ShapeDtypeStruct((1, MOTIF), _f32),
        scratch_shapes=[pltpu.VMEM((1, HIDDEN), _f32)],
    )(x, ns, deg, W1, b1, W2, b2)


def kernel(x, edge_index, W1, b1, W2, b2):
    src = edge_index[0]
    dst = edge_index[1]
    neigh_sum = jax.ops.segment_sum(x[dst], src, num_segments=N_NODES)
    deg = jax.ops.segment_sum(jnp.ones_like(src, dtype=x.dtype), src,
                              num_segments=N_NODES)
    return _tc_mlp(x, neigh_sum, deg[:, None], W1,
                   b1.reshape(1, HIDDEN), W2, b2.reshape(1, MOTIF))
